# speedup vs baseline: 1.2230x; 1.2230x over previous
import functools
import jax
import jax.numpy as jnp
from jax.experimental import pallas as pl
from jax.experimental.pallas import tpu as pltpu

N = 10000
E = 160000
D_IN = 256
H = 512
L = 3
G = 64

BN = 400
NB = N // BN


def _lin1_body(bias_ref, x_ref, w_ref, out_ref):
    out_ref[...] = jnp.dot(x_ref[...], w_ref[...],
                           preferred_element_type=jnp.float32) + bias_ref[0:1, :]


def _lin1(x, W_lin1, b_lin1):
    bias_mat = jnp.broadcast_to(b_lin1[None, :], (8, H))
    return pl.pallas_call(
        _lin1_body,
        grid=(NB,),
        in_specs=[
            pl.BlockSpec((8, H), lambda i: (0, 0)),
            pl.BlockSpec((BN, D_IN), lambda i: (i, 0)),
            pl.BlockSpec((D_IN, H), lambda i: (0, 0)),
        ],
        out_specs=pl.BlockSpec((BN, H), lambda i: (i, 0)),
        out_shape=jax.ShapeDtypeStruct((N, H), jnp.float32),
    )(bias_mat, x, W_lin1)


def _layer_body(w_ref, bias_ref, s_ref, h_ref, aggp_ref, aggne_ref, agga_ref,
                Wg_ref, Ws_ref, Wn_ref, W1_ref, W2_ref, Wgat_ref, out_ref):
    h = h_ref[...]
    aggp = aggp_ref[...]
    nsum = s_ref[:, 0:1]
    rs2 = s_ref[:, 1:2]
    invc = s_ref[:, 2:3]
    aggn = aggne_ref[...] + h * rs2
    gcn = jax.nn.relu(jnp.dot(aggn, Wg_ref[...], preferred_element_type=jnp.float32)
                      + nsum * bias_ref[0:1, :])
    sage = jax.nn.relu(jnp.dot(h, Ws_ref[...], preferred_element_type=jnp.float32)
                       + jnp.dot(aggp * invc, Wn_ref[...], preferred_element_type=jnp.float32)
                       + bias_ref[1:2, :])
    t = jax.nn.relu(jnp.dot(h + aggp, W1_ref[...], preferred_element_type=jnp.float32)
                    + bias_ref[2:3, :])
    gin = jax.nn.relu(jnp.dot(t, W2_ref[...], preferred_element_type=jnp.float32)
                      + bias_ref[3:4, :])
    gat = jax.nn.relu(jnp.dot(agga_ref[...], Wgat_ref[...], preferred_element_type=jnp.float32))
    out_ref[...] = (w_ref[0] * gcn + w_ref[1] * sage + w_ref[2] * gin + w_ref[3] * gat)


def _layer(w, smat, h, aggp, aggne, agga, Wg, bg, Ws, Wn, bs, W1, b1, W2, b2, Wgat):
    zero = jnp.zeros_like(bg)
    bias_mat = jnp.stack([bg, bs, b1, b2, zero, zero, zero, zero], axis=0)
    full = lambda i: (0, 0)
    row = lambda i: (i, 0)
    return pl.pallas_call(
        _layer_body,
        grid=(NB,),
        in_specs=[
            pl.BlockSpec(memory_space=pltpu.SMEM),
            pl.BlockSpec((8, H), full),
            pl.BlockSpec((BN, 128), row),
            pl.BlockSpec((BN, H), row),
            pl.BlockSpec((BN, H), row),
            pl.BlockSpec((BN, H), row),
            pl.BlockSpec((BN, H), row),
            pl.BlockSpec((H, H), full),
            pl.BlockSpec((H, H), full),
            pl.BlockSpec((H, H), full),
            pl.BlockSpec((H, H), full),
            pl.BlockSpec((H, H), full),
            pl.BlockSpec((H, H), full),
        ],
        out_specs=pl.BlockSpec((BN, H), row),
        out_shape=jax.ShapeDtypeStruct((N, H), jnp.float32),
    )(w, bias_mat, smat, h, aggp, aggne, agga, Wg, Ws, Wn, W1, W2, Wgat)


def _final_body(sc_ref, bias_ref, h1_ref, h2_ref, h3_ref, W1_ref, W2_ref, W3_ref, out_ref):
    j1 = sc_ref[0] * h1_ref[...]
    j2 = sc_ref[1] * h2_ref[...]
    j3 = h3_ref[...]
    mx = jnp.maximum(jnp.maximum(j1, j2), j3)
    sm = j1 + j2 + j3
    cc = (jnp.dot(j1, W1_ref[...], preferred_element_type=jnp.float32)
          + jnp.dot(j2, W2_ref[...], preferred_element_type=jnp.float32)
          + jnp.dot(j3, W3_ref[...], preferred_element_type=jnp.float32)
          + bias_ref[0:1, :])
    out_ref[...] = (sc_ref[2] * mx + (sc_ref[3] / 3.0 + sc_ref[4]) * sm + sc_ref[5] * cc)


def _final(scal, h1, h2, h3, W_la, b_la):
    bias_mat = jnp.broadcast_to(b_la[None, :], (8, H))
    Wla1, Wla2, Wla3 = W_la[0:H], W_la[H:2 * H], W_la[2 * H:3 * H]
    full = lambda i: (0, 0)
    row = lambda i: (i, 0)
    return pl.pallas_call(
        _final_body,
        grid=(NB,),
        in_specs=[
            pl.BlockSpec(memory_space=pltpu.SMEM),
            pl.BlockSpec((8, H), full),
            pl.BlockSpec((BN, H), row),
            pl.BlockSpec((BN, H), row),
            pl.BlockSpec((BN, H), row),
            pl.BlockSpec((H, H), full),
            pl.BlockSpec((H, H), full),
            pl.BlockSpec((H, H), full),
        ],
        out_specs=pl.BlockSpec((BN, H), row),
        out_shape=jax.ShapeDtypeStruct((N, H), jnp.float32),
    )(scal, bias_mat, h1, h2, h3, Wla1, Wla2, Wla3)


def kernel(x, W_lin1, b_lin1, W_gcn, b_gcn, W_sage_self, W_sage_nei, b_sage,
           W_gin1, b_gin1, W_gin2, b_gin2, W_gat, a_src, a_dst, W_la, b_la,
           na_alphas, sc_alphas, la_alphas, pool_alphas, edge_index, batch):
    src, dst = edge_index[0], edge_index[1]
    na_w = jax.nn.softmax(na_alphas, axis=-1)
    sc_w = jax.nn.softmax(sc_alphas, axis=-1)
    la_w = jax.nn.softmax(la_alphas, axis=-1)
    pool_w = jax.nn.softmax(pool_alphas, axis=-1)

    cnt = jax.ops.segment_sum(jnp.ones((E,), jnp.float32), dst, num_segments=N)
    deg = cnt + 1.0
    rs = jax.lax.rsqrt(deg)
    rs2 = rs * rs
    normv = rs[src] * rs[dst]
    nsum = jax.ops.segment_sum(normv, dst, num_segments=N) + rs2
    invc = 1.0 / jnp.maximum(cnt, 1.0)
    smat = jnp.zeros((N, 128), jnp.float32)
    smat = smat.at[:, 0].set(nsum).at[:, 1].set(rs2).at[:, 2].set(invc)

    h = _lin1(x, W_lin1, b_lin1)
    w = na_w[0]
    hs_list = []
    for i in range(L):
        v_src = W_gat[i] @ a_src[i]
        v_dst = W_gat[i] @ a_dst[i]
        s_s = h @ v_src
        s_d = h @ v_dst
        e = jax.nn.leaky_relu(s_s[src] + s_d[dst], 0.2)
        m = jax.ops.segment_max(e, dst, num_segments=N)
        m = jnp.where(jnp.isfinite(m), m, 0.0)
        ee = jnp.exp(e - m[dst])
        den = jax.ops.segment_sum(ee, dst, num_segments=N)
        alpha = ee / (den[dst] + 1e-16)

        hsrc = h[src]
        aggp = jax.ops.segment_sum(hsrc, dst, num_segments=N)
        aggne = jax.ops.segment_sum(hsrc * normv[:, None], dst, num_segments=N)
        agga = jax.ops.segment_sum(hsrc * alpha[:, None], dst, num_segments=N)

        h = _layer(w, smat, h, aggp, aggne, agga,
                   W_gcn[i], b_gcn[i], W_sage_self[i], W_sage_nei[i], b_sage[i],
                   W_gin1[i], b_gin1[i], W_gin2[i], b_gin2[i], W_gat[i])
        hs_list.append(h)

    scal = jnp.stack([sc_w[0, 1], sc_w[1, 1], la_w[0, 0], la_w[0, 1],
                      la_w[0, 2], la_w[0, 3]])
    merge = _final(scal, hs_list[0], hs_list[1], hs_list[2], W_la, b_la)

    cntb = jax.ops.segment_sum(jnp.ones((N,), jnp.float32), batch, num_segments=G)
    sump = jax.ops.segment_sum(merge, batch, num_segments=G)
    meanp = sump / jnp.maximum(cntb, 1.0)[:, None]
    maxp = jax.ops.segment_max(merge, batch, num_segments=G)
    maxp = jnp.where(cntb[:, None] > 0, maxp, 0.0)
    return pool_w[0, 0] * meanp + pool_w[0, 1] * maxp + pool_w[0, 2] * sump


# speedup vs baseline: 1.7909x; 1.4644x over previous
import functools
import jax
import jax.numpy as jnp
from jax import lax
from jax.experimental import pallas as pl
from jax.experimental.pallas import tpu as pltpu
from jax.experimental.pallas import tpu_sc as plsc

N = 10000
E = 160000
D_IN = 256
H = 512
L = 3
G = 64

BN = 400
NB = N // BN

NCORE = 2
NSUB = 16
NW = NCORE * NSUB
KB = 128
NBAT_W = 40
NBAT = NW * NBAT_W
E2 = NBAT * KB
NP = 10240
ROWS_T = NP // NSUB
FC = 128
NCHUNK = H // FC
NHALF = NP // NCORE
ACC_R = NHALF + KB
RW = NHALF // NSUB
ZR = ACC_R // NSUB
NBAT_S = NBAT // NSUB

_f32 = jnp.float32


def _mesh():
    return plsc.VectorSubcoreMesh(core_axis_name="c", subcore_axis_name="s")


def _wid():
    cid = lax.axis_index("c")
    sid = lax.axis_index("s")
    return cid, sid, sid * NCORE + cid


def _zero_1d(ref, nvec):
    z = jnp.zeros((16,), _f32)
    def body(i, _):
        ref[pl.ds(i * 16, 16)] = z
        return ()
    lax.fori_loop(0, nvec, body, ())


def _fill_1d(ref, nvec, val):
    v = jnp.full((16,), val, _f32)
    def body(i, _):
        ref[pl.ds(i * 16, 16)] = v
        return ()
    lax.fori_loop(0, nvec, body, ())


def _combine_sum(acc, sh, tmp, outv, out_hbm, cid, sid):
    pltpu.sync_copy(acc, sh.at[sid])
    plsc.subcore_barrier()
    base = sid * ROWS_T
    pltpu.sync_copy(sh.at[0, pl.ds(base, ROWS_T)], outv)
    def body(j, _):
        pltpu.sync_copy(sh.at[j, pl.ds(base, ROWS_T)], tmp)
        def add(i, _):
            outv[pl.ds(i * 16, 16)] = outv[pl.ds(i * 16, 16)] + tmp[pl.ds(i * 16, 16)]
            return ()
        lax.fori_loop(0, ROWS_T // 16, add, ())
        return ()
    lax.fori_loop(1, NSUB, body, ())
    pltpu.sync_copy(outv, out_hbm.at[cid, pl.ds(base, ROWS_T)])


def _combine_max(acc, sh, tmp, outv, out_hbm, cid, sid):
    pltpu.sync_copy(acc, sh.at[sid])
    plsc.subcore_barrier()
    base = sid * ROWS_T
    pltpu.sync_copy(sh.at[0, pl.ds(base, ROWS_T)], outv)
    def body(j, _):
        pltpu.sync_copy(sh.at[j, pl.ds(base, ROWS_T)], tmp)
        def mx(i, _):
            outv[pl.ds(i * 16, 16)] = jnp.maximum(outv[pl.ds(i * 16, 16)],
                                                  tmp[pl.ds(i * 16, 16)])
            return ()
        lax.fori_loop(0, ROWS_T // 16, mx, ())
        return ()
    lax.fori_loop(1, NSUB, body, ())
    pltpu.sync_copy(outv, out_hbm.at[cid, pl.ds(base, ROWS_T)])


@functools.partial(
    pl.kernel, mesh=_mesh(),
    compiler_params=pltpu.CompilerParams(needs_layout_passes=False),
    out_type=jax.ShapeDtypeStruct((NCORE, NP), _f32),
    scratch_types=[
        pltpu.VMEM((NBAT_W, KB), jnp.int32),
        pltpu.VMEM((NP,), _f32),
        pltpu.VMEM_SHARED((NSUB, NP), _f32),
        pltpu.VMEM((ROWS_T,), _f32),
        pltpu.VMEM((ROWS_T,), _f32),
    ],
)
def _sc_cnt(dst_hbm, out_hbm, dst_v, acc, sh, tmp, outv):
    cid, sid, wid = _wid()
    pltpu.sync_copy(dst_hbm.at[pl.ds(wid * NBAT_W, NBAT_W)], dst_v)
    _zero_1d(acc, NP // 16)
    ones = jnp.full((16,), 1.0, _f32)
    def body(b, _):
        for t in range(KB // 16):
            idx = dst_v[b, pl.ds(t * 16, 16)]
            plsc.addupdate_scatter(acc, [idx], ones)
        return ()
    lax.fori_loop(0, NBAT_W, body, ())
    _combine_sum(acc, sh, tmp, outv, out_hbm, cid, sid)


@functools.partial(
    pl.kernel, mesh=_mesh(),
    compiler_params=pltpu.CompilerParams(needs_layout_passes=False),
    out_type=jax.ShapeDtypeStruct((NCORE, NP), _f32),
    scratch_types=[
        pltpu.VMEM((NBAT_W, KB), jnp.int32),
        pltpu.VMEM((NBAT_W, KB), jnp.int32),
        pltpu.VMEM((NP,), _f32),
        pltpu.VMEM((NP,), _f32),
        pltpu.VMEM_SHARED((NSUB, NP), _f32),
        pltpu.VMEM((ROWS_T,), _f32),
        pltpu.VMEM((ROWS_T,), _f32),
    ],
)
def _sc_aggrs(src_hbm, dst_hbm, rs_hbm, out_hbm, src_v, dst_v, rs_v, acc, sh, tmp, outv):
    cid, sid, wid = _wid()
    pltpu.sync_copy(src_hbm.at[pl.ds(wid * NBAT_W, NBAT_W)], src_v)
    pltpu.sync_copy(dst_hbm.at[pl.ds(wid * NBAT_W, NBAT_W)], dst_v)
    pltpu.sync_copy(rs_hbm, rs_v)
    _zero_1d(acc, NP // 16)
    def body(b, _):
        for t in range(KB // 16):
            si = src_v[b, pl.ds(t * 16, 16)]
            di = dst_v[b, pl.ds(t * 16, 16)]
            r = plsc.load_gather(rs_v, [si])
            plsc.addupdate_scatter(acc, [di], r)
        return ()
    lax.fori_loop(0, NBAT_W, body, ())
    _combine_sum(acc, sh, tmp, outv, out_hbm, cid, sid)


@functools.partial(
    pl.kernel, mesh=_mesh(),
    compiler_params=pltpu.CompilerParams(needs_layout_passes=False),
    out_type=[jax.ShapeDtypeStruct((NBAT, KB), _f32),
              jax.ShapeDtypeStruct((NCORE, NP), _f32)],
    scratch_types=[
        pltpu.VMEM((NBAT_W, KB), jnp.int32),
        pltpu.VMEM((NBAT_W, KB), jnp.int32),
        pltpu.VMEM((NP,), _f32),
        pltpu.VMEM((NP,), _f32),
        pltpu.VMEM((NBAT_W, KB), _f32),
        pltpu.VMEM((NP,), _f32),
        pltpu.VMEM_SHARED((NSUB, NP), _f32),
        pltpu.VMEM((ROWS_T,), _f32),
        pltpu.VMEM((ROWS_T,), _f32),
    ],
)
def _sc_attn_a(src_hbm, dst_hbm, ss_hbm, sd_hbm, e_out, m_out,
               src_v, dst_v, ss_v, sd_v, ebuf, macc, sh, tmp, outv):
    cid, sid, wid = _wid()
    pltpu.sync_copy(src_hbm.at[pl.ds(wid * NBAT_W, NBAT_W)], src_v)
    pltpu.sync_copy(dst_hbm.at[pl.ds(wid * NBAT_W, NBAT_W)], dst_v)
    pltpu.sync_copy(ss_hbm, ss_v)
    pltpu.sync_copy(sd_hbm, sd_v)
    _fill_1d(macc, NP // 16, -1e30)
    lane = lax.iota(jnp.int32, 16)
    def body(b, _):
        for t in range(KB // 16):
            si = src_v[b, pl.ds(t * 16, 16)]
            di = dst_v[b, pl.ds(t * 16, 16)]
            s = plsc.load_gather(ss_v, [si]) + plsc.load_gather(sd_v, [di])
            e = jnp.where(s >= 0.0, s, 0.2 * s)
            ebuf[b, pl.ds(t * 16, 16)] = e
            for j in range(16):
                cur = plsc.load_gather(macc, [di])
                new = jnp.maximum(cur, e)
                plsc.store_scatter(macc, [di], new, mask=lane == j)
        return ()
    lax.fori_loop(0, NBAT_W, body, ())
    pltpu.sync_copy(ebuf, e_out.at[pl.ds(wid * NBAT_W, NBAT_W)])
    _combine_max(macc, sh, tmp, outv, m_out, cid, sid)


@functools.partial(
    pl.kernel, mesh=_mesh(),
    compiler_params=pltpu.CompilerParams(needs_layout_passes=False),
    out_type=[jax.ShapeDtypeStruct((NBAT, KB), _f32),
              jax.ShapeDtypeStruct((NCORE, NP), _f32)],
    scratch_types=[
        pltpu.VMEM((NBAT_W, KB), jnp.int32),
        pltpu.VMEM((NBAT_W, KB), jnp.int32),
        pltpu.VMEM((NBAT_W, KB), _f32),
        pltpu.VMEM((NBAT_W, KB), _f32),
        pltpu.VMEM((NP,), _f32),
        pltpu.VMEM((NP,), _f32),
        pltpu.VMEM((NP,), _f32),
        pltpu.VMEM_SHARED((NSUB, NP), _f32),
        pltpu.VMEM((ROWS_T,), _f32),
        pltpu.VMEM((ROWS_T,), _f32),
    ],
)
def _sc_attn_b(src_hbm, dst_hbm, e_hbm, m_hbm, sdeg_hbm, ee2_out, den_out,
               src_v, dst_v, ebuf, eebuf, m_v, sg_v, dacc, sh, tmp, outv):
    cid, sid, wid = _wid()
    pltpu.sync_copy(src_hbm.at[pl.ds(wid * NBAT_W, NBAT_W)], src_v)
    pltpu.sync_copy(dst_hbm.at[pl.ds(wid * NBAT_W, NBAT_W)], dst_v)
    pltpu.sync_copy(e_hbm.at[pl.ds(wid * NBAT_W, NBAT_W)], ebuf)
    pltpu.sync_copy(m_hbm, m_v)
    pltpu.sync_copy(sdeg_hbm, sg_v)
    _zero_1d(dacc, NP // 16)
    def body(b, _):
        for t in range(KB // 16):
            si = src_v[b, pl.ds(t * 16, 16)]
            di = dst_v[b, pl.ds(t * 16, 16)]
            e = ebuf[b, pl.ds(t * 16, 16)]
            mi = plsc.load_gather(m_v, [di])
            ee = jnp.exp(e - mi)
            plsc.addupdate_scatter(dacc, [di], ee)
            sg = plsc.load_gather(sg_v, [si])
            eebuf[b, pl.ds(t * 16, 16)] = ee * sg
        return ()
    lax.fori_loop(0, NBAT_W, body, ())
    pltpu.sync_copy(eebuf, ee2_out.at[pl.ds(wid * NBAT_W, NBAT_W)])
    _combine_sum(dacc, sh, tmp, outv, den_out, cid, sid)


def _zero_acc(acc, zbuf, zbase):
    pltpu.sync_copy(zbuf, acc.at[pl.ds(zbase, KB)])
    pltpu.sync_copy(zbuf, acc.at[pl.ds(zbase + KB, KB)])
    pltpu.sync_copy(zbuf.at[pl.ds(0, ZR - 2 * KB)],
                    acc.at[pl.ds(zbase + 2 * KB, ZR - 2 * KB)])


def _fill_zbuf(zbuf):
    z = jnp.zeros((16,), _f32)
    def zb(i, _):
        for t in range(FC // 16):
            zbuf[i, pl.ds(t * 16, 16)] = z
        return ()
    lax.fori_loop(0, KB, zb, ())


@functools.partial(
    pl.kernel, mesh=_mesh(),
    compiler_params=pltpu.CompilerParams(needs_layout_passes=False),
    out_type=jax.ShapeDtypeStruct((NCHUNK, NP, FC), _f32),
    scratch_types=[
        pltpu.VMEM((NBAT_S, KB), jnp.int32),
        pltpu.VMEM((NBAT_S, KB), jnp.int32),
        pltpu.VMEM((KB, FC), _f32),
        pltpu.VMEM((KB, FC), _f32),
        pltpu.VMEM_SHARED((ACC_R, FC), _f32),
        pltpu.SemaphoreType.DMA,
    ],
)
def _sc_agg1(src_hbm, dstl_hbm, hc_hbm, pp_out,
             src_v, dst_v, rows_h, zbuf, accp, semh):
    cid, sid, wid = _wid()
    pltpu.sync_copy(src_hbm.at[pl.ds(sid * NBAT_S, NBAT_S)], src_v)
    pltpu.sync_copy(dstl_hbm.at[cid].at[pl.ds(sid * NBAT_S, NBAT_S)], dst_v)
    _fill_zbuf(zbuf)
    zbase = sid * ZR
    obase = sid * RW
    for cc in range(NCHUNK):
        _zero_acc(accp, zbuf, zbase)
        plsc.subcore_barrier()
        def body(b, _):
            idx = src_v.at[b]
            d1 = pltpu.async_copy(hc_hbm.at[cc].at[idx], rows_h, semh)
            d1.wait()
            pltpu.sync_copy(rows_h, accp.at[dst_v.at[b]], add=True)
            return ()
        lax.fori_loop(0, NBAT_S, body, ())
        plsc.subcore_barrier()
        orow = cid * NHALF + obase
        pltpu.sync_copy(accp.at[pl.ds(obase, RW)],
                        pp_out.at[cc, pl.ds(orow, RW)])
        plsc.subcore_barrier()


@functools.partial(
    pl.kernel, mesh=_mesh(),
    compiler_params=pltpu.CompilerParams(needs_layout_passes=False),
    out_type=jax.ShapeDtypeStruct((NCHUNK, NP, FC), _f32),
    scratch_types=[
        pltpu.VMEM((NBAT_S, KB), jnp.int32),
        pltpu.VMEM((NBAT_S, KB), jnp.int32),
        pltpu.VMEM((NBAT_S, KB), _f32),
        pltpu.VMEM((KB, FC), _f32),
        pltpu.VMEM((KB, FC), _f32),
        pltpu.VMEM((KB, FC), _f32),
        pltpu.VMEM_SHARED((ACC_R, FC), _f32),
        pltpu.SemaphoreType.DMA,
    ],
)
def _sc_agg_a(src_hbm, dstl_hbm, ee2_hbm, gc_hbm, pa_out,
              src_v, dst_v, ee_v, rows_g, buf_a, zbuf, acca, semg):
    cid, sid, wid = _wid()
    pltpu.sync_copy(src_hbm.at[pl.ds(sid * NBAT_S, NBAT_S)], src_v)
    pltpu.sync_copy(dstl_hbm.at[cid].at[pl.ds(sid * NBAT_S, NBAT_S)], dst_v)
    pltpu.sync_copy(ee2_hbm.at[pl.ds(sid * NBAT_S, NBAT_S)], ee_v)
    _fill_zbuf(zbuf)
    zbase = sid * ZR
    obase = sid * RW
    for cc in range(NCHUNK):
        _zero_acc(acca, zbuf, zbase)
        plsc.subcore_barrier()
        def body(b, _):
            idx = src_v.at[b]
            d2 = pltpu.async_copy(gc_hbm.at[cc].at[idx], rows_g, semg)
            d2.wait()
            def scale(j, _):
                ee16 = ee_v[b, pl.ds(j * 16, 16)]
                for jj in range(16):
                    k = j * 16 + jj
                    w = jnp.full((16,), ee16[jj], _f32)
                    for t in range(FC // 16):
                        buf_a[k, pl.ds(t * 16, 16)] = rows_g[k, pl.ds(t * 16, 16)] * w
                return ()
            lax.fori_loop(0, KB // 16, scale, ())
            didx = dst_v.at[b]
            pltpu.sync_copy(buf_a, acca.at[didx], add=True)
            return ()
        lax.fori_loop(0, NBAT_S, body, ())
        plsc.subcore_barrier()
        orow = cid * NHALF + obase
        pltpu.sync_copy(acca.at[pl.ds(obase, RW)],
                        pa_out.at[cc, pl.ds(orow, RW)])
        plsc.subcore_barrier()


def _lin1_body(bias_ref, x_ref, w_ref, out_ref):
    out_ref[...] = jnp.dot(x_ref[...], w_ref[...],
                           preferred_element_type=jnp.float32) + bias_ref[0:1, :]


def _lin1(x, W_lin1, b_lin1):
    bias_mat = jnp.broadcast_to(b_lin1[None, :], (8, H))
    return pl.pallas_call(
        _lin1_body,
        grid=(NB,),
        in_specs=[
            pl.BlockSpec((8, H), lambda i: (0, 0)),
            pl.BlockSpec((BN, D_IN), lambda i: (i, 0)),
            pl.BlockSpec((D_IN, H), lambda i: (0, 0)),
        ],
        out_specs=pl.BlockSpec((BN, H), lambda i: (i, 0)),
        out_shape=jax.ShapeDtypeStruct((N, H), jnp.float32),
    )(bias_mat, x, W_lin1)


def _cat(ref):
    return jnp.concatenate([ref[c] for c in range(NCHUNK)], axis=-1)


def _layer_body(w_ref, bias_ref, s_ref, h_ref, pp_ref, pn_ref, pa_ref,
                Wg_ref, Ws_ref, Wn_ref, W1_ref, W2_ref, Wgat_ref, out_ref):
    h = h_ref[...]
    nsum = s_ref[:, 0:1]
    rs2 = s_ref[:, 1:2]
    invc = s_ref[:, 2:3]
    rs = s_ref[:, 3:4]
    invden = s_ref[:, 4:5]
    aggp = _cat(pp_ref)
    aggn = _cat(pn_ref) * rs + h * rs2
    agga = _cat(pa_ref) * invden
    gcn = jax.nn.relu(jnp.dot(aggn, Wg_ref[...], preferred_element_type=jnp.float32)
                      + nsum * bias_ref[0:1, :])
    sage = jax.nn.relu(jnp.dot(h, Ws_ref[...], preferred_element_type=jnp.float32)
                       + jnp.dot(aggp * invc, Wn_ref[...], preferred_element_type=jnp.float32)
                       + bias_ref[1:2, :])
    t = jax.nn.relu(jnp.dot(h + aggp, W1_ref[...], preferred_element_type=jnp.float32)
                    + bias_ref[2:3, :])
    gin = jax.nn.relu(jnp.dot(t, W2_ref[...], preferred_element_type=jnp.float32)
                      + bias_ref[3:4, :])
    gat = jax.nn.relu(jnp.dot(agga, Wgat_ref[...], preferred_element_type=jnp.float32))
    out_ref[...] = (w_ref[0] * gcn + w_ref[1] * sage + w_ref[2] * gin + w_ref[3] * gat)


def _layer(w, smat, h, pp, pn, pa, Wg, bg, Ws, Wn, bs, W1, b1, W2, b2, Wgat):
    zero = jnp.zeros_like(bg)
    bias_mat = jnp.stack([bg, bs, b1, b2, zero, zero, zero, zero], axis=0)
    full = lambda i: (0, 0)
    row = lambda i: (i, 0)
    chunk = lambda i: (0, i, 0)
    return pl.pallas_call(
        _layer_body,
        grid=(NB,),
        in_specs=[
            pl.BlockSpec(memory_space=pltpu.SMEM),
            pl.BlockSpec((8, H), full),
            pl.BlockSpec((BN, 128), row),
            pl.BlockSpec((BN, H), row),
            pl.BlockSpec((NCHUNK, BN, FC), chunk),
            pl.BlockSpec((NCHUNK, BN, FC), chunk),
            pl.BlockSpec((NCHUNK, BN, FC), chunk),
            pl.BlockSpec((H, H), full),
            pl.BlockSpec((H, H), full),
            pl.BlockSpec((H, H), full),
            pl.BlockSpec((H, H), full),
            pl.BlockSpec((H, H), full),
            pl.BlockSpec((H, H), full),
        ],
        out_specs=pl.BlockSpec((BN, H), row),
        out_shape=jax.ShapeDtypeStruct((N, H), jnp.float32),
    )(w, bias_mat, smat, h, pp, pn, pa, Wg, Ws, Wn, W1, W2, Wgat)


def _final_body(sc_ref, bias_ref, h1_ref, h2_ref, h3_ref, W1_ref, W2_ref, W3_ref, out_ref):
    j1 = sc_ref[0] * h1_ref[...]
    j2 = sc_ref[1] * h2_ref[...]
    j3 = h3_ref[...]
    mx = jnp.maximum(jnp.maximum(j1, j2), j3)
    sm = j1 + j2 + j3
    cc = (jnp.dot(j1, W1_ref[...], preferred_element_type=jnp.float32)
          + jnp.dot(j2, W2_ref[...], preferred_element_type=jnp.float32)
          + jnp.dot(j3, W3_ref[...], preferred_element_type=jnp.float32)
          + bias_ref[0:1, :])
    out_ref[...] = (sc_ref[2] * mx + (sc_ref[3] / 3.0 + sc_ref[4]) * sm + sc_ref[5] * cc)


def _final(scal, h1, h2, h3, W_la, b_la):
    bias_mat = jnp.broadcast_to(b_la[None, :], (8, H))
    Wla1, Wla2, Wla3 = W_la[0:H], W_la[H:2 * H], W_la[2 * H:3 * H]
    full = lambda i: (0, 0)
    row = lambda i: (i, 0)
    return pl.pallas_call(
        _final_body,
        grid=(NB,),
        in_specs=[
            pl.BlockSpec(memory_space=pltpu.SMEM),
            pl.BlockSpec((8, H), full),
            pl.BlockSpec((BN, H), row),
            pl.BlockSpec((BN, H), row),
            pl.BlockSpec((BN, H), row),
            pl.BlockSpec((H, H), full),
            pl.BlockSpec((H, H), full),
            pl.BlockSpec((H, H), full),
        ],
        out_specs=pl.BlockSpec((BN, H), row),
        out_shape=jax.ShapeDtypeStruct((N, H), jnp.float32),
    )(scal, bias_mat, h1, h2, h3, Wla1, Wla2, Wla3)


def _chunked(a_np):
    ap = jnp.pad(a_np, ((0, NP - N), (0, 0)))
    return jnp.stack([ap[:, c * FC:(c + 1) * FC] for c in range(NCHUNK)], axis=0)


def kernel(x, W_lin1, b_lin1, W_gcn, b_gcn, W_sage_self, W_sage_nei, b_sage,
           W_gin1, b_gin1, W_gin2, b_gin2, W_gat, a_src, a_dst, W_la, b_la,
           na_alphas, sc_alphas, la_alphas, pool_alphas, edge_index, batch):
    src, dst = edge_index[0], edge_index[1]
    na_w = jax.nn.softmax(na_alphas, axis=-1)
    sc_w = jax.nn.softmax(sc_alphas, axis=-1)
    la_w = jax.nn.softmax(la_alphas, axis=-1)
    pool_w = jax.nn.softmax(pool_alphas, axis=-1)

    srcp = jnp.concatenate([src, jnp.zeros((E2 - E,), jnp.int32)]).reshape(NBAT, KB)
    dstp = jnp.concatenate([dst, jnp.full((E2 - E,), NP - 1, jnp.int32)]).reshape(NBAT, KB)
    dstl = jnp.stack(
        [jnp.where((dstp >= c * NHALF) & (dstp < (c + 1) * NHALF),
                   dstp - c * NHALF, NHALF) for c in range(NCORE)], axis=0)

    cnt_p = _sc_cnt(dstp)
    cnt = (cnt_p[0] + cnt_p[1])[:N]
    deg = cnt + 1.0
    rs = jax.lax.rsqrt(deg)
    rs2 = rs * rs
    rs_pad = jnp.pad(rs, (0, NP - N))
    sdeg_pad = jnp.pad(jnp.sqrt(deg), (0, NP - N))
    aggrs_p = _sc_aggrs(srcp, dstp, rs_pad)
    nsum = rs * (aggrs_p[0] + aggrs_p[1])[:N] + rs2
    invc = 1.0 / jnp.maximum(cnt, 1.0)
    smat0 = jnp.zeros((N, 128), jnp.float32)
    smat0 = (smat0.at[:, 0].set(nsum).at[:, 1].set(rs2)
             .at[:, 2].set(invc).at[:, 3].set(rs))

    h = _lin1(x, W_lin1, b_lin1)
    w = na_w[0]
    hs_list = []
    for i in range(L):
        v_src = W_gat[i] @ a_src[i]
        v_dst = W_gat[i] @ a_dst[i]
        s_s = jnp.pad(h @ v_src, (0, NP - N))
        s_d = jnp.pad(h @ v_dst, (0, NP - N))
        e_arr, m_p = _sc_attn_a(srcp, dstp, s_s, s_d)
        m = jnp.maximum(m_p[0], m_p[1])
        ee2, den_p = _sc_attn_b(srcp, dstp, e_arr, m, sdeg_pad)
        den = (den_p[0] + den_p[1])[:N]
        invden = 1.0 / (den + 1e-16)

        hc = _chunked(h)
        gc = _chunked(h * rs[:, None])
        pp = _sc_agg1(srcp, dstl, hc)
        pn = _sc_agg1(srcp, dstl, gc)
        pa = _sc_agg_a(srcp, dstl, ee2, gc)

        smat = smat0.at[:, 4].set(invden)
        h = _layer(w, smat, h, pp, pn, pa,
                   W_gcn[i], b_gcn[i], W_sage_self[i], W_sage_nei[i], b_sage[i],
                   W_gin1[i], b_gin1[i], W_gin2[i], b_gin2[i], W_gat[i])
        hs_list.append(h)

    scal = jnp.stack([sc_w[0, 1], sc_w[1, 1], la_w[0, 0], la_w[0, 1],
                      la_w[0, 2], la_w[0, 3]])
    merge = _final(scal, hs_list[0], hs_list[1], hs_list[2], W_la, b_la)

    cntb = jax.ops.segment_sum(jnp.ones((N,), jnp.float32), batch, num_segments=G)
    sump = jax.ops.segment_sum(merge, batch, num_segments=G)
    meanp = sump / jnp.maximum(cntb, 1.0)[:, None]
    maxp = jax.ops.segment_max(merge, batch, num_segments=G)
    maxp = jnp.where(cntb[:, None] > 0, maxp, 0.0)
    return pool_w[0, 0] * meanp + pool_w[0, 1] * maxp + pool_w[0, 2] * sump


# speedup vs baseline: 2.0910x; 1.1675x over previous
import functools
import jax
import jax.numpy as jnp
from jax import lax
from jax.experimental import pallas as pl
from jax.experimental.pallas import tpu as pltpu
from jax.experimental.pallas import tpu_sc as plsc

N = 10000
E = 160000
D_IN = 256
H = 512
L = 3
G = 64

BN = 400
NB = N // BN

NCORE = 2
NSUB = 16
NW = NCORE * NSUB
KB = 128
NBAT_W = 40
NBAT = NW * NBAT_W
E2 = NBAT * KB
NP = 10240
ROWS_T = NP // NSUB
FC = 128
NCHUNK = H // FC
NHALF = NP // NCORE
ACC_R = NHALF + KB
RW = NHALF // NSUB
ZR = ACC_R // NSUB
NBAT_S = NBAT // NSUB

_f32 = jnp.float32


def _mesh():
    return plsc.VectorSubcoreMesh(core_axis_name="c", subcore_axis_name="s")


def _wid():
    cid = lax.axis_index("c")
    sid = lax.axis_index("s")
    return cid, sid, sid * NCORE + cid


def _zero_1d(ref, nvec):
    z = jnp.zeros((16,), _f32)
    def body(i, _):
        ref[pl.ds(i * 16, 16)] = z
        return ()
    lax.fori_loop(0, nvec, body, ())


def _fill_1d(ref, nvec, val):
    v = jnp.full((16,), val, _f32)
    def body(i, _):
        ref[pl.ds(i * 16, 16)] = v
        return ()
    lax.fori_loop(0, nvec, body, ())


def _combine_sum(acc, sh, tmp, outv, out_hbm, cid, sid):
    pltpu.sync_copy(acc, sh.at[sid])
    plsc.subcore_barrier()
    base = sid * ROWS_T
    pltpu.sync_copy(sh.at[0, pl.ds(base, ROWS_T)], outv)
    def body(j, _):
        pltpu.sync_copy(sh.at[j, pl.ds(base, ROWS_T)], tmp)
        def add(i, _):
            outv[pl.ds(i * 16, 16)] = outv[pl.ds(i * 16, 16)] + tmp[pl.ds(i * 16, 16)]
            return ()
        lax.fori_loop(0, ROWS_T // 16, add, ())
        return ()
    lax.fori_loop(1, NSUB, body, ())
    pltpu.sync_copy(outv, out_hbm.at[cid, pl.ds(base, ROWS_T)])


def _combine_max(acc, sh, tmp, outv, out_hbm, cid, sid):
    pltpu.sync_copy(acc, sh.at[sid])
    plsc.subcore_barrier()
    base = sid * ROWS_T
    pltpu.sync_copy(sh.at[0, pl.ds(base, ROWS_T)], outv)
    def body(j, _):
        pltpu.sync_copy(sh.at[j, pl.ds(base, ROWS_T)], tmp)
        def mx(i, _):
            outv[pl.ds(i * 16, 16)] = jnp.maximum(outv[pl.ds(i * 16, 16)],
                                                  tmp[pl.ds(i * 16, 16)])
            return ()
        lax.fori_loop(0, ROWS_T // 16, mx, ())
        return ()
    lax.fori_loop(1, NSUB, body, ())
    pltpu.sync_copy(outv, out_hbm.at[cid, pl.ds(base, ROWS_T)])


@functools.partial(
    pl.kernel, mesh=_mesh(),
    compiler_params=pltpu.CompilerParams(needs_layout_passes=False),
    out_type=jax.ShapeDtypeStruct((NCORE, NP), _f32),
    scratch_types=[
        pltpu.VMEM((NBAT_W, KB), jnp.int32),
        pltpu.VMEM((NP,), _f32),
        pltpu.VMEM_SHARED((NSUB, NP), _f32),
        pltpu.VMEM((ROWS_T,), _f32),
        pltpu.VMEM((ROWS_T,), _f32),
    ],
)
def _sc_cnt(dst_hbm, out_hbm, dst_v, acc, sh, tmp, outv):
    cid, sid, wid = _wid()
    pltpu.sync_copy(dst_hbm.at[pl.ds(wid * NBAT_W, NBAT_W)], dst_v)
    _zero_1d(acc, NP // 16)
    ones = jnp.full((16,), 1.0, _f32)
    def body(b, _):
        for t in range(KB // 16):
            idx = dst_v[b, pl.ds(t * 16, 16)]
            plsc.addupdate_scatter(acc, [idx], ones)
        return ()
    lax.fori_loop(0, NBAT_W, body, ())
    _combine_sum(acc, sh, tmp, outv, out_hbm, cid, sid)


@functools.partial(
    pl.kernel, mesh=_mesh(),
    compiler_params=pltpu.CompilerParams(needs_layout_passes=False),
    out_type=jax.ShapeDtypeStruct((NCORE, NP), _f32),
    scratch_types=[
        pltpu.VMEM((NBAT_W, KB), jnp.int32),
        pltpu.VMEM((NBAT_W, KB), jnp.int32),
        pltpu.VMEM((NP,), _f32),
        pltpu.VMEM((NP,), _f32),
        pltpu.VMEM_SHARED((NSUB, NP), _f32),
        pltpu.VMEM((ROWS_T,), _f32),
        pltpu.VMEM((ROWS_T,), _f32),
    ],
)
def _sc_aggrs(src_hbm, dst_hbm, rs_hbm, out_hbm, src_v, dst_v, rs_v, acc, sh, tmp, outv):
    cid, sid, wid = _wid()
    pltpu.sync_copy(src_hbm.at[pl.ds(wid * NBAT_W, NBAT_W)], src_v)
    pltpu.sync_copy(dst_hbm.at[pl.ds(wid * NBAT_W, NBAT_W)], dst_v)
    pltpu.sync_copy(rs_hbm, rs_v)
    _zero_1d(acc, NP // 16)
    def body(b, _):
        for t in range(KB // 16):
            si = src_v[b, pl.ds(t * 16, 16)]
            di = dst_v[b, pl.ds(t * 16, 16)]
            r = plsc.load_gather(rs_v, [si])
            plsc.addupdate_scatter(acc, [di], r)
        return ()
    lax.fori_loop(0, NBAT_W, body, ())
    _combine_sum(acc, sh, tmp, outv, out_hbm, cid, sid)


@functools.partial(
    pl.kernel, mesh=_mesh(),
    compiler_params=pltpu.CompilerParams(needs_layout_passes=False),
    out_type=[jax.ShapeDtypeStruct((NBAT, KB), _f32),
              jax.ShapeDtypeStruct((NCORE, NP), _f32)],
    scratch_types=[
        pltpu.VMEM((NBAT_W, KB), jnp.int32),
        pltpu.VMEM((NBAT_W, KB), jnp.int32),
        pltpu.VMEM((NP,), _f32),
        pltpu.VMEM((NP,), _f32),
        pltpu.VMEM((NBAT_W, KB), _f32),
        pltpu.VMEM((NP,), _f32),
        pltpu.VMEM_SHARED((NSUB, NP), _f32),
        pltpu.VMEM((ROWS_T,), _f32),
        pltpu.VMEM((ROWS_T,), _f32),
    ],
)
def _sc_attn_a(src_hbm, dst_hbm, ss_hbm, sd_hbm, e_out, m_out,
               src_v, dst_v, ss_v, sd_v, ebuf, macc, sh, tmp, outv):
    cid, sid, wid = _wid()
    pltpu.sync_copy(src_hbm.at[pl.ds(wid * NBAT_W, NBAT_W)], src_v)
    pltpu.sync_copy(dst_hbm.at[pl.ds(wid * NBAT_W, NBAT_W)], dst_v)
    pltpu.sync_copy(ss_hbm, ss_v)
    pltpu.sync_copy(sd_hbm, sd_v)
    _fill_1d(macc, NP // 16, -1e30)
    def body(b, _):
        for t in range(KB // 16):
            si = src_v[b, pl.ds(t * 16, 16)]
            di = dst_v[b, pl.ds(t * 16, 16)]
            s = plsc.load_gather(ss_v, [si]) + plsc.load_gather(sd_v, [di])
            e = jnp.where(s >= 0.0, s, 0.2 * s)
            ebuf[b, pl.ds(t * 16, 16)] = e
            cur = plsc.load_gather(macc, [di])
            plsc.store_scatter(macc, [di], jnp.maximum(cur, e))
        return ()
    lax.fori_loop(0, NBAT_W, body, ())
    pltpu.sync_copy(ebuf, e_out.at[pl.ds(wid * NBAT_W, NBAT_W)])
    _combine_max(macc, sh, tmp, outv, m_out, cid, sid)


@functools.partial(
    pl.kernel, mesh=_mesh(),
    compiler_params=pltpu.CompilerParams(needs_layout_passes=False),
    out_type=[jax.ShapeDtypeStruct((NBAT, KB), _f32),
              jax.ShapeDtypeStruct((NCORE, NP), _f32)],
    scratch_types=[
        pltpu.VMEM((NBAT_W, KB), jnp.int32),
        pltpu.VMEM((NBAT_W, KB), jnp.int32),
        pltpu.VMEM((NBAT_W, KB), _f32),
        pltpu.VMEM((NBAT_W, KB), _f32),
        pltpu.VMEM((NP,), _f32),
        pltpu.VMEM((NP,), _f32),
        pltpu.VMEM((NP,), _f32),
        pltpu.VMEM_SHARED((NSUB, NP), _f32),
        pltpu.VMEM((ROWS_T,), _f32),
        pltpu.VMEM((ROWS_T,), _f32),
    ],
)
def _sc_attn_b(src_hbm, dst_hbm, e_hbm, m_hbm, sdeg_hbm, ee2_out, den_out,
               src_v, dst_v, ebuf, eebuf, m_v, sg_v, dacc, sh, tmp, outv):
    cid, sid, wid = _wid()
    pltpu.sync_copy(src_hbm.at[pl.ds(wid * NBAT_W, NBAT_W)], src_v)
    pltpu.sync_copy(dst_hbm.at[pl.ds(wid * NBAT_W, NBAT_W)], dst_v)
    pltpu.sync_copy(e_hbm.at[pl.ds(wid * NBAT_W, NBAT_W)], ebuf)
    pltpu.sync_copy(m_hbm, m_v)
    pltpu.sync_copy(sdeg_hbm, sg_v)
    _zero_1d(dacc, NP // 16)
    def body(b, _):
        for t in range(KB // 16):
            si = src_v[b, pl.ds(t * 16, 16)]
            di = dst_v[b, pl.ds(t * 16, 16)]
            e = ebuf[b, pl.ds(t * 16, 16)]
            mi = plsc.load_gather(m_v, [di])
            ee = jnp.exp(jnp.minimum(e - mi, 80.0))
            plsc.addupdate_scatter(dacc, [di], ee)
            sg = plsc.load_gather(sg_v, [si])
            eebuf[b, pl.ds(t * 16, 16)] = ee * sg
        return ()
    lax.fori_loop(0, NBAT_W, body, ())
    pltpu.sync_copy(eebuf, ee2_out.at[pl.ds(wid * NBAT_W, NBAT_W)])
    _combine_sum(dacc, sh, tmp, outv, den_out, cid, sid)


def _zero_acc(acc, zbuf, zbase):
    pltpu.sync_copy(zbuf, acc.at[pl.ds(zbase, KB)])
    pltpu.sync_copy(zbuf, acc.at[pl.ds(zbase + KB, KB)])
    pltpu.sync_copy(zbuf.at[pl.ds(0, ZR - 2 * KB)],
                    acc.at[pl.ds(zbase + 2 * KB, ZR - 2 * KB)])


def _fill_zbuf(zbuf):
    z = jnp.zeros((16,), _f32)
    def zb(i, _):
        for t in range(FC // 16):
            zbuf[i, pl.ds(t * 16, 16)] = z
        return ()
    lax.fori_loop(0, KB, zb, ())


@functools.partial(
    pl.kernel, mesh=_mesh(),
    compiler_params=pltpu.CompilerParams(needs_layout_passes=False),
    out_type=jax.ShapeDtypeStruct((NCHUNK, NP, FC), _f32),
    scratch_types=[
        pltpu.VMEM((NBAT_S, KB), jnp.int32),
        pltpu.VMEM((NBAT_S, KB), jnp.int32),
        pltpu.VMEM((KB, FC), _f32),
        pltpu.VMEM((KB, FC), _f32),
        pltpu.VMEM_SHARED((ACC_R, FC), _f32),
        pltpu.SemaphoreType.DMA,
    ],
)
def _sc_agg1(src_hbm, dstl_hbm, hc_hbm, pp_out,
             src_v, dst_v, rows_h, zbuf, accp, semh):
    cid, sid, wid = _wid()
    pltpu.sync_copy(src_hbm.at[pl.ds(sid * NBAT_S, NBAT_S)], src_v)
    pltpu.sync_copy(dstl_hbm.at[cid].at[pl.ds(sid * NBAT_S, NBAT_S)], dst_v)
    _fill_zbuf(zbuf)
    zbase = sid * ZR
    obase = sid * RW
    for cc in range(NCHUNK):
        _zero_acc(accp, zbuf, zbase)
        plsc.subcore_barrier()
        def body(b, _):
            idx = src_v.at[b]
            d1 = pltpu.async_copy(hc_hbm.at[cc].at[idx], rows_h, semh)
            d1.wait()
            pltpu.sync_copy(rows_h, accp.at[dst_v.at[b]], add=True)
            return ()
        lax.fori_loop(0, NBAT_S, body, ())
        plsc.subcore_barrier()
        orow = cid * NHALF + obase
        pltpu.sync_copy(accp.at[pl.ds(obase, RW)],
                        pp_out.at[cc, pl.ds(orow, RW)])
        plsc.subcore_barrier()


@functools.partial(
    pl.kernel, mesh=_mesh(),
    compiler_params=pltpu.CompilerParams(needs_layout_passes=False),
    out_type=jax.ShapeDtypeStruct((NCHUNK, NP, FC), _f32),
    scratch_types=[
        pltpu.VMEM((NBAT_S, KB), jnp.int32),
        pltpu.VMEM((NBAT_S, KB), jnp.int32),
        pltpu.VMEM((NBAT_S, KB), _f32),
        pltpu.VMEM((KB, FC), _f32),
        pltpu.VMEM((KB, FC), _f32),
        pltpu.VMEM((KB, FC), _f32),
        pltpu.VMEM_SHARED((ACC_R, FC), _f32),
        pltpu.SemaphoreType.DMA,
    ],
)
def _sc_agg_a(src_hbm, dstl_hbm, ee2_hbm, gc_hbm, pa_out,
              src_v, dst_v, ee_v, rows_g, buf_a, zbuf, acca, semg):
    cid, sid, wid = _wid()
    pltpu.sync_copy(src_hbm.at[pl.ds(sid * NBAT_S, NBAT_S)], src_v)
    pltpu.sync_copy(dstl_hbm.at[cid].at[pl.ds(sid * NBAT_S, NBAT_S)], dst_v)
    pltpu.sync_copy(ee2_hbm.at[pl.ds(sid * NBAT_S, NBAT_S)], ee_v)
    _fill_zbuf(zbuf)
    zbase = sid * ZR
    obase = sid * RW
    for cc in range(NCHUNK):
        _zero_acc(acca, zbuf, zbase)
        plsc.subcore_barrier()
        def body(b, _):
            idx = src_v.at[b]
            d2 = pltpu.async_copy(gc_hbm.at[cc].at[idx], rows_g, semg)
            d2.wait()
            def scale(j, _):
                ee16 = ee_v[b, pl.ds(j * 16, 16)]
                for jj in range(16):
                    k = j * 16 + jj
                    w = jnp.full((16,), ee16[jj], _f32)
                    for t in range(FC // 16):
                        buf_a[k, pl.ds(t * 16, 16)] = rows_g[k, pl.ds(t * 16, 16)] * w
                return ()
            lax.fori_loop(0, KB // 16, scale, ())
            didx = dst_v.at[b]
            pltpu.sync_copy(buf_a, acca.at[didx], add=True)
            return ()
        lax.fori_loop(0, NBAT_S, body, ())
        plsc.subcore_barrier()
        orow = cid * NHALF + obase
        pltpu.sync_copy(acca.at[pl.ds(obase, RW)],
                        pa_out.at[cc, pl.ds(orow, RW)])
        plsc.subcore_barrier()


def _lin1_body(bias_ref, x_ref, w_ref, out_ref):
    out_ref[...] = jnp.dot(x_ref[...], w_ref[...],
                           preferred_element_type=jnp.float32) + bias_ref[0:1, :]


def _lin1(x, W_lin1, b_lin1):
    bias_mat = jnp.broadcast_to(b_lin1[None, :], (8, H))
    return pl.pallas_call(
        _lin1_body,
        grid=(NB,),
        in_specs=[
            pl.BlockSpec((8, H), lambda i: (0, 0)),
            pl.BlockSpec((BN, D_IN), lambda i: (i, 0)),
            pl.BlockSpec((D_IN, H), lambda i: (0, 0)),
        ],
        out_specs=pl.BlockSpec((BN, H), lambda i: (i, 0)),
        out_shape=jax.ShapeDtypeStruct((N, H), jnp.float32),
    )(bias_mat, x, W_lin1)


def _cat(ref):
    return jnp.concatenate([ref[c] for c in range(NCHUNK)], axis=-1)


def _layer_body(w_ref, bias_ref, s_ref, h_ref, pp_ref, pn_ref, pa_ref,
                Wg_ref, Ws_ref, Wn_ref, W1_ref, W2_ref, Wgat_ref, out_ref):
    h = h_ref[...]
    nsum = s_ref[:, 0:1]
    rs2 = s_ref[:, 1:2]
    invc = s_ref[:, 2:3]
    rs = s_ref[:, 3:4]
    invden = s_ref[:, 4:5]
    aggp = _cat(pp_ref)
    aggn = _cat(pn_ref) * rs + h * rs2
    agga = _cat(pa_ref) * invden
    gcn = jax.nn.relu(jnp.dot(aggn, Wg_ref[...], preferred_element_type=jnp.float32)
                      + nsum * bias_ref[0:1, :])
    sage = jax.nn.relu(jnp.dot(h, Ws_ref[...], preferred_element_type=jnp.float32)
                       + jnp.dot(aggp * invc, Wn_ref[...], preferred_element_type=jnp.float32)
                       + bias_ref[1:2, :])
    t = jax.nn.relu(jnp.dot(h + aggp, W1_ref[...], preferred_element_type=jnp.float32)
                    + bias_ref[2:3, :])
    gin = jax.nn.relu(jnp.dot(t, W2_ref[...], preferred_element_type=jnp.float32)
                      + bias_ref[3:4, :])
    gat = jax.nn.relu(jnp.dot(agga, Wgat_ref[...], preferred_element_type=jnp.float32))
    out_ref[...] = (w_ref[0] * gcn + w_ref[1] * sage + w_ref[2] * gin + w_ref[3] * gat)


def _layer(w, smat, h, pp, pn, pa, Wg, bg, Ws, Wn, bs, W1, b1, W2, b2, Wgat):
    zero = jnp.zeros_like(bg)
    bias_mat = jnp.stack([bg, bs, b1, b2, zero, zero, zero, zero], axis=0)
    full = lambda i: (0, 0)
    row = lambda i: (i, 0)
    chunk = lambda i: (0, i, 0)
    return pl.pallas_call(
        _layer_body,
        grid=(NB,),
        in_specs=[
            pl.BlockSpec(memory_space=pltpu.SMEM),
            pl.BlockSpec((8, H), full),
            pl.BlockSpec((BN, 128), row),
            pl.BlockSpec((BN, H), row),
            pl.BlockSpec((NCHUNK, BN, FC), chunk),
            pl.BlockSpec((NCHUNK, BN, FC), chunk),
            pl.BlockSpec((NCHUNK, BN, FC), chunk),
            pl.BlockSpec((H, H), full),
            pl.BlockSpec((H, H), full),
            pl.BlockSpec((H, H), full),
            pl.BlockSpec((H, H), full),
            pl.BlockSpec((H, H), full),
            pl.BlockSpec((H, H), full),
        ],
        out_specs=pl.BlockSpec((BN, H), row),
        out_shape=jax.ShapeDtypeStruct((N, H), jnp.float32),
    )(w, bias_mat, smat, h, pp, pn, pa, Wg, Ws, Wn, W1, W2, Wgat)


def _final_body(sc_ref, bias_ref, h1_ref, h2_ref, h3_ref, W1_ref, W2_ref, W3_ref, out_ref):
    j1 = sc_ref[0] * h1_ref[...]
    j2 = sc_ref[1] * h2_ref[...]
    j3 = h3_ref[...]
    mx = jnp.maximum(jnp.maximum(j1, j2), j3)
    sm = j1 + j2 + j3
    cc = (jnp.dot(j1, W1_ref[...], preferred_element_type=jnp.float32)
          + jnp.dot(j2, W2_ref[...], preferred_element_type=jnp.float32)
          + jnp.dot(j3, W3_ref[...], preferred_element_type=jnp.float32)
          + bias_ref[0:1, :])
    out_ref[...] = (sc_ref[2] * mx + (sc_ref[3] / 3.0 + sc_ref[4]) * sm + sc_ref[5] * cc)


def _final(scal, h1, h2, h3, W_la, b_la):
    bias_mat = jnp.broadcast_to(b_la[None, :], (8, H))
    Wla1, Wla2, Wla3 = W_la[0:H], W_la[H:2 * H], W_la[2 * H:3 * H]
    full = lambda i: (0, 0)
    row = lambda i: (i, 0)
    return pl.pallas_call(
        _final_body,
        grid=(NB,),
        in_specs=[
            pl.BlockSpec(memory_space=pltpu.SMEM),
            pl.BlockSpec((8, H), full),
            pl.BlockSpec((BN, H), row),
            pl.BlockSpec((BN, H), row),
            pl.BlockSpec((BN, H), row),
            pl.BlockSpec((H, H), full),
            pl.BlockSpec((H, H), full),
            pl.BlockSpec((H, H), full),
        ],
        out_specs=pl.BlockSpec((BN, H), row),
        out_shape=jax.ShapeDtypeStruct((N, H), jnp.float32),
    )(scal, bias_mat, h1, h2, h3, Wla1, Wla2, Wla3)


def _chunked(a_np):
    ap = jnp.pad(a_np, ((0, NP - N), (0, 0)))
    return jnp.stack([ap[:, c * FC:(c + 1) * FC] for c in range(NCHUNK)], axis=0)


def kernel(x, W_lin1, b_lin1, W_gcn, b_gcn, W_sage_self, W_sage_nei, b_sage,
           W_gin1, b_gin1, W_gin2, b_gin2, W_gat, a_src, a_dst, W_la, b_la,
           na_alphas, sc_alphas, la_alphas, pool_alphas, edge_index, batch):
    src, dst = edge_index[0], edge_index[1]
    na_w = jax.nn.softmax(na_alphas, axis=-1)
    sc_w = jax.nn.softmax(sc_alphas, axis=-1)
    la_w = jax.nn.softmax(la_alphas, axis=-1)
    pool_w = jax.nn.softmax(pool_alphas, axis=-1)

    srcf = jnp.concatenate([src, jnp.zeros((E2 - E,), jnp.int32)])
    dstf = jnp.concatenate([dst, jnp.full((E2 - E,), NP - 1, jnp.int32)])
    perm = jnp.argsort(dstf)
    srcp = srcf[perm].reshape(16, E2 // 16).T.reshape(NBAT, KB)
    dstp = dstf[perm].reshape(16, E2 // 16).T.reshape(NBAT, KB)
    dstl = jnp.stack(
        [jnp.where((dstp >= c * NHALF) & (dstp < (c + 1) * NHALF),
                   dstp - c * NHALF, NHALF) for c in range(NCORE)], axis=0)

    cnt_p = _sc_cnt(dstp)
    cnt = (cnt_p[0] + cnt_p[1])[:N]
    deg = cnt + 1.0
    rs = jax.lax.rsqrt(deg)
    rs2 = rs * rs
    rs_pad = jnp.pad(rs, (0, NP - N))
    sdeg_pad = jnp.pad(jnp.sqrt(deg), (0, NP - N))
    aggrs_p = _sc_aggrs(srcp, dstp, rs_pad)
    nsum = rs * (aggrs_p[0] + aggrs_p[1])[:N] + rs2
    invc = 1.0 / jnp.maximum(cnt, 1.0)
    smat0 = jnp.zeros((N, 128), jnp.float32)
    smat0 = (smat0.at[:, 0].set(nsum).at[:, 1].set(rs2)
             .at[:, 2].set(invc).at[:, 3].set(rs))

    h = _lin1(x, W_lin1, b_lin1)
    w = na_w[0]
    hs_list = []
    for i in range(L):
        v_src = W_gat[i] @ a_src[i]
        v_dst = W_gat[i] @ a_dst[i]
        s_s = jnp.pad(h @ v_src, (0, NP - N))
        s_d = jnp.pad(h @ v_dst, (0, NP - N))
        e_arr, m_p = _sc_attn_a(srcp, dstp, s_s, s_d)
        m = jnp.maximum(m_p[0], m_p[1])
        ee2, den_p = _sc_attn_b(srcp, dstp, e_arr, m, sdeg_pad)
        den = (den_p[0] + den_p[1])[:N]
        invden = 1.0 / (den + 1e-16)

        hc = _chunked(h)
        gc = _chunked(h * rs[:, None])
        pp = _sc_agg1(srcp, dstl, hc)
        pn = _sc_agg1(srcp, dstl, gc)
        pa = _sc_agg_a(srcp, dstl, ee2, gc)

        smat = smat0.at[:, 4].set(invden)
        h = _layer(w, smat, h, pp, pn, pa,
                   W_gcn[i], b_gcn[i], W_sage_self[i], W_sage_nei[i], b_sage[i],
                   W_gin1[i], b_gin1[i], W_gin2[i], b_gin2[i], W_gat[i])
        hs_list.append(h)

    scal = jnp.stack([sc_w[0, 1], sc_w[1, 1], la_w[0, 0], la_w[0, 1],
                      la_w[0, 2], la_w[0, 3]])
    merge = _final(scal, hs_list[0], hs_list[1], hs_list[2], W_la, b_la)

    cntb = jax.ops.segment_sum(jnp.ones((N,), jnp.float32), batch, num_segments=G)
    sump = jax.ops.segment_sum(merge, batch, num_segments=G)
    meanp = sump / jnp.maximum(cntb, 1.0)[:, None]
    maxp = jax.ops.segment_max(merge, batch, num_segments=G)
    maxp = jnp.where(cntb[:, None] > 0, maxp, 0.0)
    return pool_w[0, 0] * meanp + pool_w[0, 1] * maxp + pool_w[0, 2] * sump
